# baseline (speedup 1.0000x reference)
import functools

import jax
import jax.numpy as jnp
from jax import lax
from jax.experimental import pallas as pl
from jax.experimental.pallas import tpu as pltpu
from jax.experimental.pallas import tpu_sc as plsc

TOPK = 5
NROWS = 4096
NCOLS = 100000
LANES = 16
NWORKERS = 32
ROWS_PER_W = NROWS // NWORKERS
BATCH = 64
BCC = 1000
BRS = 4096
CT = 52000
CCH = 400
NCH = (NCOLS - CT) // CCH


def _sc_gather_body(xt_hbm, y_hbm, v_hbm, y_v, tiles_v, v_v, sem):
    wid = lax.axis_index("s") * 2 + lax.axis_index("c")
    base = pl.multiple_of(wid * ROWS_PER_W, ROWS_PER_W)
    pltpu.sync_copy(y_hbm.at[pl.ds(base, ROWS_PER_W)], y_v)
    iota = lax.iota(jnp.int32, LANES)
    for b in range(ROWS_PER_W // BATCH):
        for j in range(BATCH):
            lane = b * BATCH + j
            y_s = y_v[pl.ds((lane // LANES) * LANES, LANES)][lane % LANES]
            c8 = pl.multiple_of(y_s & ~7, 8)
            pltpu.async_copy(xt_hbm.at[pl.ds(c8, 8), pl.ds(base, ROWS_PER_W)],
                             tiles_v.at[j], sem)
        for j in range(BATCH):
            pltpu.make_async_copy(
                xt_hbm.at[pl.ds(0, 8), pl.ds(0, ROWS_PER_W)],
                tiles_v.at[j], sem).wait()
        for g in range(BATCH // LANES):
            grp = jnp.zeros((LANES,), jnp.float32)
            for k in range(LANES):
                j = g * LANES + k
                lane = b * BATCH + j
                y_s = y_v[pl.ds((lane // LANES) * LANES, LANES)][lane % LANES]
                s8 = y_s & 7
                vec = tiles_v[j, s8, pl.ds((lane // LANES) * LANES, LANES)]
                grp = jnp.where(iota == k, vec, grp)
            v_v[pl.ds(b * BATCH + g * LANES, LANES)] = grp
    pltpu.sync_copy(v_v, v_hbm.at[pl.ds(base, ROWS_PER_W)])


@functools.cache
def _sc_gather_kernel():
    return pl.kernel(
        _sc_gather_body,
        mesh=plsc.VectorSubcoreMesh(core_axis_name="c", subcore_axis_name="s"),
        out_type=jax.ShapeDtypeStruct((NROWS,), jnp.float32),
        scratch_types=[
            pltpu.VMEM((ROWS_PER_W,), jnp.int32),
            pltpu.VMEM((BATCH, 8, ROWS_PER_W), jnp.float32),
            pltpu.VMEM((ROWS_PER_W,), jnp.float32),
            pltpu.SemaphoreType.DMA,
        ],
    )


def _sc_count_body(xt_hbm, y_hbm, v_hbm, cnt_hbm, y_v, v_v, cnt_v,
                   buf0, buf1, sem0, sem1):
    wid = lax.axis_index("s") * 2 + lax.axis_index("c")
    base = pl.multiple_of(wid * ROWS_PER_W, ROWS_PER_W)
    pltpu.sync_copy(y_hbm.at[pl.ds(base, ROWS_PER_W)], y_v)
    pltpu.sync_copy(v_hbm.at[pl.ds(base, ROWS_PER_W)], v_v)
    bufs = (buf0, buf1)
    sems = (sem0, sem1)
    y_g = [y_v[pl.ds(g * LANES, LANES)] for g in range(ROWS_PER_W // LANES)]
    v_g = [v_v[pl.ds(g * LANES, LANES)] for g in range(ROWS_PER_W // LANES)]
    acc = [jnp.zeros((LANES,), jnp.int32) for _ in range(ROWS_PER_W // LANES)]

    def start(t):
        return pltpu.async_copy(
            xt_hbm.at[pl.ds(pl.multiple_of(CT + t * CCH, 8), CCH),
                      pl.ds(base, ROWS_PER_W)],
            bufs[t % 2], sems[t % 2])

    cp = start(0)
    for t in range(NCH):
        if t + 1 < NCH:
            nxt = start(t + 1)
        cp.wait()
        buf = bufs[t % 2]

        def cls(c, carry):
            col = jnp.full((LANES,), CT + t * CCH + c, jnp.int32)
            out = []
            for g in range(ROWS_PER_W // LANES):
                x = buf[c, pl.ds(g * LANES, LANES)]
                m = (x > v_g[g]) | ((x == v_g[g]) & (col < y_g[g]))
                out.append(carry[g] + jnp.where(m, 1, 0))
            return tuple(out)

        acc = lax.fori_loop(0, CCH, cls, tuple(acc))
        acc = list(acc)
        if t + 1 < NCH:
            cp = nxt
    for g in range(ROWS_PER_W // LANES):
        cnt_v[pl.ds(g * LANES, LANES)] = acc[g]
    pltpu.sync_copy(cnt_v, cnt_hbm.at[pl.ds(base, ROWS_PER_W)])


@functools.cache
def _sc_count_kernel():
    return pl.kernel(
        _sc_count_body,
        mesh=plsc.VectorSubcoreMesh(core_axis_name="c", subcore_axis_name="s"),
        out_type=jax.ShapeDtypeStruct((NROWS,), jnp.int32),
        scratch_types=[
            pltpu.VMEM((ROWS_PER_W,), jnp.int32),
            pltpu.VMEM((ROWS_PER_W,), jnp.float32),
            pltpu.VMEM((ROWS_PER_W,), jnp.int32),
            pltpu.VMEM((CCH, ROWS_PER_W), jnp.float32),
            pltpu.VMEM((CCH, ROWS_PER_W), jnp.float32),
            pltpu.SemaphoreType.DMA,
            pltpu.SemaphoreType.DMA,
        ],
    )


def _combine_body(r_ref, c_ref, out_ref):
    rank = r_ref[...] + c_ref[...].astype(jnp.float32)
    match = jnp.where(rank < (TOPK - 0.5), 1.0, 0.0)
    out_ref[...] = jnp.sum(match, keepdims=True).reshape(1, 1) * (1.0 / NROWS)


def _tc_count_body(x_ref, v_ref, y_ref, out_ref, acc_ref):
    j = pl.program_id(0)

    @pl.when(j == 0)
    def _init():
        acc_ref[...] = jnp.zeros_like(acc_ref)

    x = x_ref[...]
    v = v_ref[...]
    yv = y_ref[...]
    cols = j * BCC + lax.broadcasted_iota(jnp.int32, (BCC, BRS), 0)
    m = (x > v) | ((x == v) & (cols < yv))
    ones = jnp.where(m, 1.0, 0.0)
    acc_ref[...] += ones.reshape(BCC // 8, 8, BRS).sum(axis=0)

    @pl.when(j == pl.num_programs(0) - 1)
    def _fin():
        out_ref[...] = acc_ref[...].sum(axis=0, keepdims=True)


def kernel(y_hat_logits, y):
    y32 = y.astype(jnp.int32)
    xt = y_hat_logits.T
    v1 = _sc_gather_kernel()(xt, y32)
    cnt_sc = _sc_count_kernel()(xt, y32, v1)
    rank_tc = pl.pallas_call(
        _tc_count_body,
        grid=(CT // BCC,),
        in_specs=[
            pl.BlockSpec((BCC, BRS), lambda j: (j, 0)),
            pl.BlockSpec((1, BRS), lambda j: (0, 0)),
            pl.BlockSpec((1, BRS), lambda j: (0, 0)),
        ],
        out_specs=pl.BlockSpec((1, BRS), lambda j: (0, 0)),
        out_shape=jax.ShapeDtypeStruct((1, BRS), jnp.float32),
        scratch_shapes=[pltpu.VMEM((8, BRS), jnp.float32)],
    )(xt, v1.reshape(1, NROWS), y32.reshape(1, NROWS))
    out = pl.pallas_call(
        _combine_body,
        in_specs=[
            pl.BlockSpec((1, NROWS), lambda: (0, 0)),
            pl.BlockSpec((1, NROWS), lambda: (0, 0)),
        ],
        out_specs=pl.BlockSpec((1, 1), lambda: (0, 0)),
        out_shape=jax.ShapeDtypeStruct((1, 1), jnp.float32),
    )(rank_tc, cnt_sc.reshape(1, NROWS))
    return out[0, 0]


# speedup vs baseline: 1.1055x; 1.1055x over previous
import functools

import jax
import jax.numpy as jnp
from jax import lax
from jax.experimental import pallas as pl
from jax.experimental.pallas import tpu as pltpu
from jax.experimental.pallas import tpu_sc as plsc

TOPK = 5
NROWS = 4096
NCOLS = 100000
LANES = 16
NWORKERS = 32
ROWS_PER_W = NROWS // NWORKERS
BATCH = 64
BCC = 1000
BRS = 4096
CT = 60000
CCH = 400
NCH = (NCOLS - CT) // CCH


def _sc_gather_body(xt_hbm, y_hbm, v_hbm, y_v, tiles_v, v_v, sem):
    wid = lax.axis_index("s") * 2 + lax.axis_index("c")
    base = pl.multiple_of(wid * ROWS_PER_W, ROWS_PER_W)
    pltpu.sync_copy(y_hbm.at[pl.ds(base, ROWS_PER_W)], y_v)
    iota = lax.iota(jnp.int32, LANES)
    for b in range(ROWS_PER_W // BATCH):
        for j in range(BATCH):
            lane = b * BATCH + j
            y_s = y_v[pl.ds((lane // LANES) * LANES, LANES)][lane % LANES]
            c8 = pl.multiple_of(y_s & ~7, 8)
            pltpu.async_copy(xt_hbm.at[pl.ds(c8, 8), pl.ds(base, ROWS_PER_W)],
                             tiles_v.at[j], sem)
        for j in range(BATCH):
            pltpu.make_async_copy(
                xt_hbm.at[pl.ds(0, 8), pl.ds(0, ROWS_PER_W)],
                tiles_v.at[j], sem).wait()
        for g in range(BATCH // LANES):
            grp = jnp.zeros((LANES,), jnp.float32)
            for k in range(LANES):
                j = g * LANES + k
                lane = b * BATCH + j
                y_s = y_v[pl.ds((lane // LANES) * LANES, LANES)][lane % LANES]
                s8 = y_s & 7
                vec = tiles_v[j, s8, pl.ds((lane // LANES) * LANES, LANES)]
                grp = jnp.where(iota == k, vec, grp)
            v_v[pl.ds(b * BATCH + g * LANES, LANES)] = grp
    pltpu.sync_copy(v_v, v_hbm.at[pl.ds(base, ROWS_PER_W)])


@functools.cache
def _sc_gather_kernel():
    return pl.kernel(
        _sc_gather_body,
        mesh=plsc.VectorSubcoreMesh(core_axis_name="c", subcore_axis_name="s"),
        out_type=jax.ShapeDtypeStruct((NROWS,), jnp.float32),
        scratch_types=[
            pltpu.VMEM((ROWS_PER_W,), jnp.int32),
            pltpu.VMEM((BATCH, 8, ROWS_PER_W), jnp.float32),
            pltpu.VMEM((ROWS_PER_W,), jnp.float32),
            pltpu.SemaphoreType.DMA,
        ],
    )


def _sc_count_body(xt_hbm, y_hbm, v_hbm, cnt_hbm, y_v, v_v, cnt_v,
                   buf0, buf1, sem0, sem1):
    wid = lax.axis_index("s") * 2 + lax.axis_index("c")
    base = pl.multiple_of(wid * ROWS_PER_W, ROWS_PER_W)
    pltpu.sync_copy(y_hbm.at[pl.ds(base, ROWS_PER_W)], y_v)
    pltpu.sync_copy(v_hbm.at[pl.ds(base, ROWS_PER_W)], v_v)
    bufs = (buf0, buf1)
    sems = (sem0, sem1)
    y_g = [y_v[pl.ds(g * LANES, LANES)] for g in range(ROWS_PER_W // LANES)]
    v_g = [v_v[pl.ds(g * LANES, LANES)] for g in range(ROWS_PER_W // LANES)]
    acc = [jnp.zeros((LANES,), jnp.int32) for _ in range(ROWS_PER_W // LANES)]

    def start(t):
        return pltpu.async_copy(
            xt_hbm.at[pl.ds(pl.multiple_of(CT + t * CCH, 8), CCH),
                      pl.ds(base, ROWS_PER_W)],
            bufs[t % 2], sems[t % 2])

    cp = start(0)
    for t in range(NCH):
        if t + 1 < NCH:
            nxt = start(t + 1)
        cp.wait()
        buf = bufs[t % 2]

        def cls(c, carry):
            col = jnp.full((LANES,), CT + t * CCH + c, jnp.int32)
            out = []
            for g in range(ROWS_PER_W // LANES):
                x = buf[c, pl.ds(g * LANES, LANES)]
                m = (x > v_g[g]) | ((x == v_g[g]) & (col < y_g[g]))
                out.append(carry[g] + jnp.where(m, 1, 0))
            return tuple(out)

        acc = lax.fori_loop(0, CCH, cls, tuple(acc))
        acc = list(acc)
        if t + 1 < NCH:
            cp = nxt
    for g in range(ROWS_PER_W // LANES):
        cnt_v[pl.ds(g * LANES, LANES)] = acc[g]
    pltpu.sync_copy(cnt_v, cnt_hbm.at[pl.ds(base, ROWS_PER_W)])


@functools.cache
def _sc_count_kernel():
    return pl.kernel(
        _sc_count_body,
        mesh=plsc.VectorSubcoreMesh(core_axis_name="c", subcore_axis_name="s"),
        out_type=jax.ShapeDtypeStruct((NROWS,), jnp.int32),
        scratch_types=[
            pltpu.VMEM((ROWS_PER_W,), jnp.int32),
            pltpu.VMEM((ROWS_PER_W,), jnp.float32),
            pltpu.VMEM((ROWS_PER_W,), jnp.int32),
            pltpu.VMEM((CCH, ROWS_PER_W), jnp.float32),
            pltpu.VMEM((CCH, ROWS_PER_W), jnp.float32),
            pltpu.SemaphoreType.DMA,
            pltpu.SemaphoreType.DMA,
        ],
    )


def _combine_body(r_ref, c_ref, out_ref):
    rank = r_ref[...] + c_ref[...].astype(jnp.float32)
    match = jnp.where(rank < (TOPK - 0.5), 1.0, 0.0)
    out_ref[...] = jnp.sum(match, keepdims=True).reshape(1, 1) * (1.0 / NROWS)


def _tc_count_body(x_ref, v_ref, y_ref, out_ref, acc_ref):
    j = pl.program_id(0)

    @pl.when(j == 0)
    def _init():
        acc_ref[...] = jnp.zeros_like(acc_ref)

    x = x_ref[...]
    v = v_ref[...]
    yv = y_ref[...]
    cols = j * BCC + lax.broadcasted_iota(jnp.int32, (BCC, BRS), 0)
    m = (x > v) | ((x == v) & (cols < yv))
    ones = jnp.where(m, 1.0, 0.0)
    acc_ref[...] += ones.reshape(BCC // 8, 8, BRS).sum(axis=0)

    @pl.when(j == pl.num_programs(0) - 1)
    def _fin():
        out_ref[...] = acc_ref[...].sum(axis=0, keepdims=True)


def kernel(y_hat_logits, y):
    y32 = y.astype(jnp.int32)
    xt = y_hat_logits.T
    v1 = _sc_gather_kernel()(xt, y32)
    cnt_sc = _sc_count_kernel()(xt, y32, v1)
    rank_tc = pl.pallas_call(
        _tc_count_body,
        grid=(CT // BCC,),
        in_specs=[
            pl.BlockSpec((BCC, BRS), lambda j: (j, 0)),
            pl.BlockSpec((1, BRS), lambda j: (0, 0)),
            pl.BlockSpec((1, BRS), lambda j: (0, 0)),
        ],
        out_specs=pl.BlockSpec((1, BRS), lambda j: (0, 0)),
        out_shape=jax.ShapeDtypeStruct((1, BRS), jnp.float32),
        scratch_shapes=[pltpu.VMEM((8, BRS), jnp.float32)],
    )(xt, v1.reshape(1, NROWS), y32.reshape(1, NROWS))
    out = pl.pallas_call(
        _combine_body,
        in_specs=[
            pl.BlockSpec((1, NROWS), lambda: (0, 0)),
            pl.BlockSpec((1, NROWS), lambda: (0, 0)),
        ],
        out_specs=pl.BlockSpec((1, 1), lambda: (0, 0)),
        out_shape=jax.ShapeDtypeStruct((1, 1), jnp.float32),
    )(rank_tc, cnt_sc.reshape(1, NROWS))
    return out[0, 0]
